# baseline (speedup 1.0000x reference)
import jax
import jax.numpy as jnp
from jax.experimental import pallas as pl


def _stage1_body(hg_up_ref, rhs_ref, users_ref, w_ref, b_ref, out_ref):
    a = jnp.dot(hg_up_ref[...], rhs_ref[...], preferred_element_type=jnp.float32)
    d = a.shape[1] // 3
    g = a[:, :d]
    s = a[:, d:2 * d]
    p = a[:, 2 * d:]
    gs = g * s
    gp = g * p
    sp = s * p
    gsp = gs * p
    msg = jnp.concatenate([g, s, p, gs, gp, sp, gsp], axis=1)
    me = jnp.dot(msg, w_ref[...], preferred_element_type=jnp.float32) + b_ref[...]
    u = users_ref[...]
    out_ref[...] = me + u + me * u


def _stage2_body(hg_pu_ref, hg_ref, init_ref, out_ref):
    out_ref[...] = init_ref[...] + jnp.dot(
        hg_pu_ref[...], hg_ref[...], preferred_element_type=jnp.float32)


def kernel(init_pois_embs, geo_pois_embs, seq_pois_embs, users_embs,
           HG_up, HG_pu, W_fusion, b_fusion):
    P, D = init_pois_embs.shape
    U = users_embs.shape[0]

    rhs = jnp.concatenate([geo_pois_embs, seq_pois_embs, init_pois_embs], axis=1)
    b2d = b_fusion.reshape(1, D)

    BU = 200
    hg = pl.pallas_call(
        _stage1_body,
        grid=(U // BU,),
        in_specs=[
            pl.BlockSpec((BU, P), lambda i: (i, 0)),
            pl.BlockSpec((P, 3 * D), lambda i: (0, 0)),
            pl.BlockSpec((BU, D), lambda i: (i, 0)),
            pl.BlockSpec((7 * D, D), lambda i: (0, 0)),
            pl.BlockSpec((1, D), lambda i: (0, 0)),
        ],
        out_specs=pl.BlockSpec((BU, D), lambda i: (i, 0)),
        out_shape=jax.ShapeDtypeStruct((U, D), jnp.float32),
    )(HG_up, rhs, users_embs, W_fusion, b2d)

    BP = 400
    out = pl.pallas_call(
        _stage2_body,
        grid=(P // BP,),
        in_specs=[
            pl.BlockSpec((BP, U), lambda i: (i, 0)),
            pl.BlockSpec((U, D), lambda i: (0, 0)),
            pl.BlockSpec((BP, D), lambda i: (i, 0)),
        ],
        out_specs=pl.BlockSpec((BP, D), lambda i: (i, 0)),
        out_shape=jax.ShapeDtypeStruct((P, D), jnp.float32),
    )(HG_pu, hg, init_pois_embs)

    return out
